# baseline (speedup 1.0000x reference)
import jax
import jax.numpy as jnp
from jax.experimental import pallas as pl
from jax.experimental.pallas import tpu as pltpu


_C = (1.0000046491622925, -0.4999035894870758, 0.33256760239601135,
      -0.2539786398410797, 0.22070538997650146, -0.14338473975658417)
_LN2 = 0.6931471805599453
_SQRTH_BITS = 0x3F3504F3


def _fastlog(u):
    bits = jax.lax.bitcast_convert_type(u, jnp.int32)
    x = bits + (0x3F800000 - _SQRTH_BITS)
    k = (x >> 23) - 127
    m = jax.lax.bitcast_convert_type((x & 0x007FFFFF) + _SQRTH_BITS, jnp.float32)
    f = m - 1.0
    q = jnp.float32(_C[5])
    for c in (_C[4], _C[3], _C[2], _C[1], _C[0]):
        q = q * f + jnp.float32(c)
    return k.astype(jnp.float32) * _LN2 + f * q


def _body(l_ref, u_ref, o_ref, w_ref):
    e = jnp.exp(2.0 * l_ref[0])
    u = u_ref[0]
    t = _fastlog(u)
    w = 1.0 / (t * t)
    w_ref[...] = w
    s = jnp.sum(w * e, axis=1, keepdims=True)
    m = jnp.max(w_ref[...] * (1.0 / s), axis=0, keepdims=True)
    o_ref[0] = e * m


def kernel(logits, uniform):
    B, K, D = uniform.shape
    out = pl.pallas_call(
        _body,
        grid=(B,),
        in_specs=[
            pl.BlockSpec((1, 1, D), lambda b: (b, 0, 0)),
            pl.BlockSpec((1, K, D), lambda b: (b, 0, 0)),
        ],
        out_specs=pl.BlockSpec((1, 1, D), lambda b: (b, 0, 0)),
        out_shape=jax.ShapeDtypeStruct((B, 1, D), jnp.float32),
        scratch_shapes=[pltpu.VMEM((K, D), jnp.float32)],
    )(logits.reshape(B, 1, D), uniform)
    return out.reshape(B, D)


# speedup vs baseline: 1.9006x; 1.9006x over previous
import jax
import jax.numpy as jnp
from jax.experimental import pallas as pl
from jax.experimental.pallas import tpu as pltpu


_C = (1.0000046491622925, -0.4999035894870758, 0.33256760239601135,
      -0.2539786398410797, 0.22070538997650146, -0.14338473975658417)
_LN2 = 0.6931471805599453
_SQRTH_BITS = 0x3F3504F3


def _fastlog(u):
    bits = jax.lax.bitcast_convert_type(u, jnp.int32)
    x = bits + (0x3F800000 - _SQRTH_BITS)
    k = (x >> 23) - 127
    m = jax.lax.bitcast_convert_type((x & 0x007FFFFF) + _SQRTH_BITS, jnp.float32)
    f = m - 1.0
    q = jnp.float32(_C[5])
    for c in (_C[4], _C[3], _C[2], _C[1], _C[0]):
        q = q * f + jnp.float32(c)
    return k.astype(jnp.float32) * _LN2 + f * q


def _body(l_ref, u_ref, o_ref, r_ref):
    e = jnp.exp(2.0 * l_ref[0])
    q = jnp.log2(u_ref[0])
    r = (1.0 / (q * q)) * e
    r_ref[...] = r
    s = jnp.sum(r, axis=1, keepdims=True)
    o_ref[0] = jnp.max(r_ref[...] * (1.0 / s), axis=0, keepdims=True)


def kernel(logits, uniform):
    B, K, D = uniform.shape
    out = pl.pallas_call(
        _body,
        grid=(B,),
        in_specs=[
            pl.BlockSpec((1, 1, D), lambda b: (b, 0, 0)),
            pl.BlockSpec((1, K, D), lambda b: (b, 0, 0)),
        ],
        out_specs=pl.BlockSpec((1, 1, D), lambda b: (b, 0, 0)),
        out_shape=jax.ShapeDtypeStruct((B, 1, D), jnp.float32),
        scratch_shapes=[pltpu.VMEM((K, D), jnp.float32)],
    )(logits.reshape(B, 1, D), uniform)
    return out.reshape(B, D)


# speedup vs baseline: 2.4299x; 1.2785x over previous
import jax
import jax.numpy as jnp
from jax.experimental import pallas as pl
from jax.experimental.pallas import tpu as pltpu

_BPB = 4


def _body(l_ref, u_ref, o_ref, r_ref):
    K = u_ref.shape[1]
    for j in range(_BPB):
        e = jnp.exp(2.0 * l_ref[j])
        t = jnp.log(u_ref[j])
        r = (1.0 / (t * t)) * e
        rj = r_ref.at[pl.ds(j * K, K), :]
        rj[...] = r
        s = jnp.sum(r, axis=1, keepdims=True)
        o_ref[j] = jnp.max(rj[...] * (1.0 / s), axis=0, keepdims=True)


def kernel(logits, uniform):
    B, K, D = uniform.shape
    out = pl.pallas_call(
        _body,
        grid=(B // _BPB,),
        in_specs=[
            pl.BlockSpec((_BPB, 1, D), lambda b: (b, 0, 0)),
            pl.BlockSpec((_BPB, K, D), lambda b: (b, 0, 0)),
        ],
        out_specs=pl.BlockSpec((_BPB, 1, D), lambda b: (b, 0, 0)),
        out_shape=jax.ShapeDtypeStruct((B, 1, D), jnp.float32),
        scratch_shapes=[pltpu.VMEM((_BPB * K, D), jnp.float32)],
    )(logits.reshape(B, 1, D), uniform)
    return out.reshape(B, D)


# speedup vs baseline: 2.4300x; 1.0001x over previous
import jax
import jax.numpy as jnp
from jax.experimental import pallas as pl
from jax.experimental.pallas import tpu as pltpu

_BPB = 4


def _body(l_ref, u_ref, o_ref, r_ref):
    K = u_ref.shape[1]
    for j in range(_BPB):
        e = jnp.exp(2.0 * l_ref[j])
        t = jnp.log(u_ref[j])
        r = (1.0 / (t * t)) * e
        rj = r_ref.at[pl.ds(j * K, K), :]
        rj[...] = r
        s = jnp.sum(r, axis=1, keepdims=True)
        o_ref[j] = jnp.max(rj[...] * (1.0 / s), axis=0, keepdims=True)


def kernel(logits, uniform):
    B, K, D = uniform.shape
    out = pl.pallas_call(
        _body,
        grid=(B // _BPB,),
        in_specs=[
            pl.BlockSpec((_BPB, 1, D), lambda b: (b, 0, 0)),
            pl.BlockSpec((_BPB, K, D), lambda b: (b, 0, 0)),
        ],
        out_specs=pl.BlockSpec((_BPB, 1, D), lambda b: (b, 0, 0)),
        out_shape=jax.ShapeDtypeStruct((B, 1, D), jnp.float32),
        scratch_shapes=[pltpu.VMEM((_BPB * K, D), jnp.float32)],
        compiler_params=pltpu.CompilerParams(
            vmem_limit_bytes=60 * 1024 * 1024,
        ),
    )(logits.reshape(B, 1, D), uniform)
    return out.reshape(B, D)
